# Initial kernel scaffold; baseline (speedup 1.0000x reference)
import jax
import jax.numpy as jnp
from jax.experimental import pallas as pl


def kernel(input_ids, embed_tokens):
    raise NotImplementedError("write your pallas kernel here")


# speedup vs baseline: 1.5756x; 1.5756x over previous
import functools
import math

import jax
import jax.numpy as jnp
import numpy as np
from jax import lax
from jax.experimental import pallas as pl
from jax.experimental.pallas import tpu as pltpu
from jax.experimental.pallas import tpu_sc as plsc

_VOCAB = 151936
_HIDDEN = 896
_HEAD_DIM = 64
_ROPE_THETA = 1000000.0
_BATCH = 4
_SEQ = 8192

_NC = 2
_NS = 16
_NW = _NC * _NS

_N_IDS = _BATCH * _SEQ
_PER_W = _N_IDS // _NW
_CHUNK = 128
_NCH = _PER_W // _CHUNK


def _gather_body(table_hbm, ids_hbm, out_hbm, idx_v, row_v, sem):
    wid = lax.axis_index("s") * _NC + lax.axis_index("c")
    base = wid * _PER_W
    pltpu.sync_copy(ids_hbm.at[pl.ds(base, _PER_W)], idx_v)

    def body(i, carry):
        off = pl.multiple_of(i * _CHUNK, _CHUNK)
        pltpu.async_copy(
            table_hbm.at[idx_v.at[pl.ds(off, _CHUNK)]], row_v, sem
        ).wait()
        pltpu.sync_copy(row_v, out_hbm.at[pl.ds(base + off, _CHUNK)])
        return carry

    lax.fori_loop(0, _NCH, body, 0)


_gather = pl.kernel(
    _gather_body,
    mesh=plsc.VectorSubcoreMesh(core_axis_name="c", subcore_axis_name="s"),
    out_type=jax.ShapeDtypeStruct((_N_IDS, _HIDDEN), jnp.float32),
    scratch_types=[
        pltpu.VMEM((_PER_W,), jnp.int32),
        pltpu.VMEM((_CHUNK, _HIDDEN), jnp.float32),
        pltpu.SemaphoreType.DMA,
    ],
)


def _rope_body(invf_ref, cos_ref, sin_ref):
    pos = lax.broadcasted_iota(jnp.int32, (_SEQ, _HEAD_DIM), 0).astype(jnp.float32)
    ang = pos * invf_ref[...]
    cos_ref[...] = jnp.cos(ang)
    sin_ref[...] = jnp.sin(ang)


_rope = pl.pallas_call(
    _rope_body,
    out_shape=(
        jax.ShapeDtypeStruct((_SEQ, _HEAD_DIM), jnp.float32),
        jax.ShapeDtypeStruct((_SEQ, _HEAD_DIM), jnp.float32),
    ),
)

_INV_FREQ = 1.0 / (_ROPE_THETA ** (np.arange(0, _HEAD_DIM, 2, dtype=np.float64) / _HEAD_DIM))
_INV_FULL = np.concatenate([_INV_FREQ, _INV_FREQ]).astype(np.float32)[None, :]


def kernel(input_ids, embed_tokens):
    ids_flat = input_ids.reshape(-1)
    hidden = _gather(embed_tokens, ids_flat).reshape(_BATCH, _SEQ, _HIDDEN)
    cos, sin = _rope(jnp.asarray(_INV_FULL))
    return hidden, cos[None], sin[None]


# speedup vs baseline: 1.6200x; 1.0282x over previous
import functools
import math

import jax
import jax.numpy as jnp
import numpy as np
from jax import lax
from jax.experimental import pallas as pl
from jax.experimental.pallas import tpu as pltpu
from jax.experimental.pallas import tpu_sc as plsc

_VOCAB = 151936
_HIDDEN = 896
_HEAD_DIM = 64
_ROPE_THETA = 1000000.0
_BATCH = 4
_SEQ = 8192

_NC = 2
_NS = 16
_NW = _NC * _NS

_N_IDS = _BATCH * _SEQ
_PER_W = _N_IDS // _NW
_CHUNK = 64
_NCH = _PER_W // _CHUNK


def _gather_body(table_hbm, ids_hbm, out_hbm, idx_v, buf0, buf1, sem0, sem1):
    wid = lax.axis_index("s") * _NC + lax.axis_index("c")
    base = wid * _PER_W
    pltpu.sync_copy(ids_hbm.at[pl.ds(base, _PER_W)], idx_v)

    def start(chunk, buf, sem):
        off = pl.multiple_of(chunk * _CHUNK, _CHUNK)
        return pltpu.async_copy(table_hbm.at[idx_v.at[pl.ds(off, _CHUNK)]], buf, sem)

    def flush(chunk, buf):
        off = pl.multiple_of(chunk * _CHUNK, _CHUNK)
        pltpu.sync_copy(buf, out_hbm.at[pl.ds(base + off, _CHUNK)])

    def wait(buf, sem):
        pltpu.make_async_copy(table_hbm.at[idx_v.at[pl.ds(0, _CHUNK)]], buf, sem).wait()

    start(0, buf0, sem0)
    start(1, buf1, sem1)

    def body(j, carry):
        c = j * 2
        wait(buf0, sem0)
        flush(c, buf0)
        start(c + 2, buf0, sem0)
        wait(buf1, sem1)
        flush(c + 1, buf1)
        start(c + 3, buf1, sem1)
        return carry

    lax.fori_loop(0, _NCH // 2 - 1, body, 0)
    wait(buf0, sem0)
    flush(_NCH - 2, buf0)
    wait(buf1, sem1)
    flush(_NCH - 1, buf1)


_gather = pl.kernel(
    _gather_body,
    mesh=plsc.VectorSubcoreMesh(core_axis_name="c", subcore_axis_name="s"),
    out_type=jax.ShapeDtypeStruct((_N_IDS, _HIDDEN), jnp.float32),
    scratch_types=[
        pltpu.VMEM((_PER_W,), jnp.int32),
        pltpu.VMEM((_CHUNK, _HIDDEN), jnp.float32),
        pltpu.VMEM((_CHUNK, _HIDDEN), jnp.float32),
        pltpu.SemaphoreType.DMA,
        pltpu.SemaphoreType.DMA,
    ],
)


def _rope_body(invf_ref, cos_ref, sin_ref):
    pos = lax.broadcasted_iota(jnp.int32, (_SEQ, _HEAD_DIM), 0).astype(jnp.float32)
    ang = pos * invf_ref[...]
    cos_ref[...] = jnp.cos(ang)
    sin_ref[...] = jnp.sin(ang)


_rope = pl.pallas_call(
    _rope_body,
    out_shape=(
        jax.ShapeDtypeStruct((_SEQ, _HEAD_DIM), jnp.float32),
        jax.ShapeDtypeStruct((_SEQ, _HEAD_DIM), jnp.float32),
    ),
)

_INV_FREQ = 1.0 / (_ROPE_THETA ** (np.arange(0, _HEAD_DIM, 2, dtype=np.float64) / _HEAD_DIM))
_INV_FULL = np.concatenate([_INV_FREQ, _INV_FREQ]).astype(np.float32)[None, :]


def kernel(input_ids, embed_tokens):
    ids_flat = input_ids.reshape(-1)
    hidden = _gather(embed_tokens, ids_flat).reshape(_BATCH, _SEQ, _HIDDEN)
    cos, sin = _rope(jnp.asarray(_INV_FULL))
    return hidden, cos[None], sin[None]


# speedup vs baseline: 1.6463x; 1.0162x over previous
import functools
import math

import jax
import jax.numpy as jnp
import numpy as np
from jax import lax
from jax.experimental import pallas as pl
from jax.experimental.pallas import tpu as pltpu
from jax.experimental.pallas import tpu_sc as plsc

_VOCAB = 151936
_HIDDEN = 896
_HEAD_DIM = 64
_ROPE_THETA = 1000000.0
_BATCH = 4
_SEQ = 8192

_NC = 2
_NS = 16
_NW = _NC * _NS

_N_IDS = _BATCH * _SEQ
_PER_W = _N_IDS // _NW
_CHUNK = 64
_NCH = _PER_W // _CHUNK


def _gather_body(table_hbm, ids_hbm, out_hbm, idx_v, buf0, buf1, sem0, sem1):
    wid = lax.axis_index("s") * _NC + lax.axis_index("c")
    base = wid * _PER_W
    pltpu.sync_copy(ids_hbm.at[pl.ds(base, _PER_W)], idx_v)

    def start(chunk, buf, sem):
        off = pl.multiple_of(chunk * _CHUNK, _CHUNK)
        return pltpu.async_copy(table_hbm.at[idx_v.at[pl.ds(off, _CHUNK)]], buf, sem)

    def flush(chunk, buf):
        off = pl.multiple_of(chunk * _CHUNK, _CHUNK)
        pltpu.sync_copy(buf, out_hbm.at[pl.ds(base + off, _CHUNK)])

    def wait(buf, sem):
        pltpu.make_async_copy(table_hbm.at[idx_v.at[pl.ds(0, _CHUNK)]], buf, sem).wait()

    start(0, buf0, sem0)
    start(1, buf1, sem1)

    def body(j, carry):
        c = j * 2
        wait(buf0, sem0)
        flush(c, buf0)
        start(c + 2, buf0, sem0)
        wait(buf1, sem1)
        flush(c + 1, buf1)
        start(c + 3, buf1, sem1)
        return carry

    lax.fori_loop(0, _NCH // 2 - 1, body, 0)
    wait(buf0, sem0)
    flush(_NCH - 2, buf0)
    wait(buf1, sem1)
    flush(_NCH - 1, buf1)


_gather = pl.kernel(
    _gather_body,
    mesh=plsc.VectorSubcoreMesh(core_axis_name="c", subcore_axis_name="s"),
    out_type=jax.ShapeDtypeStruct((_N_IDS, _HIDDEN), jnp.float32),
    scratch_types=[
        pltpu.VMEM((_PER_W,), jnp.int32),
        pltpu.VMEM((_CHUNK, _HIDDEN), jnp.float32),
        pltpu.VMEM((_CHUNK, _HIDDEN), jnp.float32),
        pltpu.SemaphoreType.DMA,
        pltpu.SemaphoreType.DMA,
    ],
)


_SHI = 128
_SLO = 64

_INV_FREQ = 1.0 / (_ROPE_THETA ** (np.arange(0, _HEAD_DIM, 2, dtype=np.float64) / _HEAD_DIM))
_W = np.concatenate([_INV_FREQ, _INV_FREQ])
_HI_ANG = (np.arange(_SHI, dtype=np.float64) * _SLO)[:, None, None] * _W[None, None, :]
_LO_ANG = np.arange(_SLO, dtype=np.float64)[None, :, None] * _W[None, None, :]
_COS_HI = np.cos(_HI_ANG).astype(np.float32)
_SIN_HI = np.sin(_HI_ANG).astype(np.float32)
_COS_LO = np.cos(_LO_ANG).astype(np.float32)
_SIN_LO = np.sin(_LO_ANG).astype(np.float32)


def _rope_body(ch, sh, cl, sl, cos_ref, sin_ref):
    chv, shv, clv, slv = ch[...], sh[...], cl[...], sl[...]
    cos_ref[...] = chv * clv - shv * slv
    sin_ref[...] = shv * clv + chv * slv


_rope = pl.pallas_call(
    _rope_body,
    out_shape=(
        jax.ShapeDtypeStruct((_SHI, _SLO, _HEAD_DIM), jnp.float32),
        jax.ShapeDtypeStruct((_SHI, _SLO, _HEAD_DIM), jnp.float32),
    ),
)


def kernel(input_ids, embed_tokens):
    ids_flat = input_ids.reshape(-1)
    hidden = _gather(embed_tokens, ids_flat).reshape(_BATCH, _SEQ, _HIDDEN)
    cos, sin = _rope(
        jnp.asarray(_COS_HI), jnp.asarray(_SIN_HI),
        jnp.asarray(_COS_LO), jnp.asarray(_SIN_LO),
    )
    return (
        hidden,
        cos.reshape(1, _SEQ, _HEAD_DIM),
        sin.reshape(1, _SEQ, _HEAD_DIM),
    )
